# baseline (speedup 1.0000x reference)
import functools

import jax
import jax.numpy as jnp
import numpy as np
from jax import lax
from jax.experimental import pallas as pl
from jax.experimental.pallas import tpu as pltpu
from jax.experimental.pallas import tpu_sc as plsc

B = 16384
V = 1000000
D = 64
NEG = 5

NC = 2
NS = 16
NW = NC * NS
L = 16

BPW = B // NW
C = 128
R = BPW // C
NV = D // L

TW = 2048
TBF = V // TW
TTAIL = V - TBF * TW
TB = TBF + 1
VP = TB * TW // 2
MAIN = TBF * TW
MP = MAIN // 2
T1 = MAIN + 512
P2 = MP + 256


def _transpose_block(in_hbm, tp_ref, out_ref, vin, sem):
    i = pl.program_id(0)

    @pl.when(i < TBF)
    def _main():
        cp = pltpu.make_async_copy(in_hbm.at[:, pl.ds(i * TW, TW)], vin, sem)
        cp.start()
        cp.wait()
        x = vin[...]
        y = jnp.transpose(x, (1, 0))
        out_ref[:, pl.ds(0, D)] = y[0:TW // 2]
        out_ref[:, pl.ds(D, D)] = y[TW // 2:TW]

    @pl.when(i == TBF)
    def _tail():
        cp = pltpu.make_async_copy(
            in_hbm.at[:, pl.ds(TBF * TW, 512)], vin.at[:, pl.ds(0, 512)], sem)
        cp.start()
        cp.wait()
        x = vin[:, pl.ds(0, 512)]
        y = jnp.transpose(x, (1, 0))
        out_ref[pl.ds(0, 256), pl.ds(0, D)] = y[0:256]
        out_ref[pl.ds(0, 256), pl.ds(D, D)] = y[256:512]
        out_ref[pl.ds(256, 32), :] = tp_ref[...]


def _to_pair_table(table_t, tail_pairs):
    return pl.pallas_call(
        _transpose_block,
        grid=(TB,),
        in_specs=[
            pl.BlockSpec(memory_space=pl.ANY),
            pl.BlockSpec((32, 2 * D), lambda i: (0, 0)),
        ],
        out_specs=pl.BlockSpec((TW // 2, 2 * D), lambda i: (i, 0)),
        out_shape=jax.ShapeDtypeStruct((VP, 2 * D), jnp.float32),
        scratch_shapes=[
            pltpu.VMEM((D, TW), jnp.float32),
            pltpu.SemaphoreType.DMA,
        ],
    )(table_t, tail_pairs)


@functools.partial(
    pl.kernel,
    out_type=[
        jax.ShapeDtypeStruct((B,), jnp.float32),
        jax.ShapeDtypeStruct((NEG * B,), jnp.float32),
    ],
    mesh=plsc.VectorSubcoreMesh(core_axis_name="c", subcore_axis_name="s"),
    compiler_params=pltpu.CompilerParams(needs_layout_passes=False),
    scratch_types=[
        pltpu.VMEM((C,), jnp.int32),
        pltpu.VMEM((C,), jnp.int32),
        pltpu.VMEM((C * NEG,), jnp.int32),
        pltpu.VMEM((C,), jnp.int32),
        pltpu.VMEM((C,), jnp.int32),
        pltpu.VMEM((C * NEG,), jnp.int32),
        pltpu.VMEM((C,), jnp.int32),
        pltpu.VMEM((C,), jnp.int32),
        pltpu.VMEM((C * NEG,), jnp.int32),
        pltpu.VMEM((C, 2 * D), jnp.float32),
        pltpu.VMEM((C, 2 * D), jnp.float32),
        pltpu.VMEM((C * NEG, 2 * D), jnp.float32),
        pltpu.VMEM((C,), jnp.float32),
        pltpu.VMEM((NEG, C), jnp.float32),
        pltpu.SemaphoreType.DMA,
    ],
)
def _sc_scores(pos_u, pos_v, neg_flat, u_pair, v_pair, out_pos, out_neg,
               idx_u, idx_v, idx_n, pair_u, pair_v, pair_n,
               off_u, off_v, off_n, u_rows, v_rows, n_rows,
               acc_p, acc_n, sem):
    wid = lax.axis_index("s") * NC + lax.axis_index("c")
    iota = lax.iota(jnp.int32, L)

    def round_body(r, _):
        base = wid * BPW + r * C
        pltpu.sync_copy(pos_u.at[pl.ds(base, C)], idx_u)
        pltpu.sync_copy(pos_v.at[pl.ds(base, C)], idx_v)
        pltpu.sync_copy(neg_flat.at[pl.ds(base * NEG, C * NEG)], idx_n)

        def pairsplit(i):
            pm = ((i >> 11) << 10) | (i & 1023)
            l = i - MAIN
            q = i - T1
            pp = jnp.where(i < MAIN, pm,
                           jnp.where(i < T1, MP + (l & 255), P2 + (q & 31)))
            ob = jnp.where(i < MAIN, (i >> 10) & 1,
                           jnp.where(i < T1, (l >> 8) & 1, (q >> 5) & 1))
            return pp, ob * D

        def shift_body(t, _):
            sl = pl.ds(t * L, L)
            pair_u[sl], off_u[sl] = pairsplit(idx_u[sl])
            pair_v[sl], off_v[sl] = pairsplit(idx_v[sl])
            return 0

        lax.fori_loop(0, C // L, shift_body, 0)

        def shift_n_body(t, _):
            sl = pl.ds(t * L, L)
            pair_n[sl], off_n[sl] = pairsplit(idx_n[sl])
            return 0

        lax.fori_loop(0, C * NEG // L, shift_n_body, 0)

        cu = pltpu.async_copy(u_pair.at[pair_u], u_rows, sem)
        cv = pltpu.async_copy(v_pair.at[pair_v], v_rows, sem)
        cn = pltpu.async_copy(v_pair.at[pair_n], n_rows, sem)
        cu.wait()
        cv.wait()
        cn.wait()

        def dot_bcast(u, w):
            p = u[0] * w[0]
            for kk in range(1, NV):
                p = p + u[kk] * w[kk]
            return jnp.full((L,), jnp.sum(p), jnp.float32)

        def load_row(ref, row, off):
            return [ref[row, pl.ds(off + L * kk, L)] for kk in range(NV)]

        def gbody(g, _):
            ou = off_u[pl.ds(g * L, L)]
            ov = off_v[pl.ds(g * L, L)]
            onn = [off_n[pl.ds(g * L * NEG + L * m, L)] for m in range(NEG)]
            accs = [jnp.zeros((L,), jnp.float32)] * (1 + NEG)
            for j in range(L):
                i = g * L + j
                lane = jnp.equal(iota, j)
                u = load_row(u_rows, i, ou[j])
                v = load_row(v_rows, i, ov[j])
                accs[0] = jnp.where(lane, dot_bcast(u, v), accs[0])
                for n in range(NEG):
                    k = j * NEG + n
                    w = load_row(n_rows, i * NEG + n, onn[k // L][k % L])
                    accs[1 + n] = jnp.where(lane, dot_bcast(u, w),
                                            accs[1 + n])
            acc_p[pl.ds(g * L, L)] = accs[0]
            for n in range(NEG):
                acc_n[n, pl.ds(g * L, L)] = accs[1 + n]
            return 0

        lax.fori_loop(0, C // L, gbody, 0)

        pltpu.sync_copy(acc_p, out_pos.at[pl.ds(base, C)])
        for n in range(NEG):
            pltpu.sync_copy(acc_n.at[n], out_neg.at[pl.ds(n * B + base, C)])
        return 0

    lax.fori_loop(0, R, round_body, 0)


def _tc_loss_kernel(p_ref, n_ref, o_ref):
    p = jnp.clip(p_ref[...], -10.0, 10.0)
    n = jnp.clip(n_ref[...], -10.0, 10.0)
    loss_pos = jnp.log1p(jnp.exp(-p))
    loss_neg = jnp.log1p(jnp.exp(n))
    o_ref[0, 0] = (jnp.sum(loss_pos) + jnp.sum(loss_neg)) * np.float32(1.0 / B)


def kernel(pos_u, pos_v, neg_v, u_table, v_table):
    pos_u = pos_u.astype(jnp.int32)
    pos_v = pos_v.astype(jnp.int32)
    neg_flat = neg_v.reshape(-1).astype(jnp.int32)

    ut = u_table[T1:]
    vt = v_table[T1:]
    u_pair = _to_pair_table(
        u_table.T, jnp.concatenate([ut[0:32], ut[32:64]], axis=1))
    v_pair = _to_pair_table(
        v_table.T, jnp.concatenate([vt[0:32], vt[32:64]], axis=1))

    dots_pos, dots_neg = _sc_scores(pos_u, pos_v, neg_flat, u_pair, v_pair)

    out = pl.pallas_call(
        _tc_loss_kernel,
        out_shape=jax.ShapeDtypeStruct((1, 1), jnp.float32),
        out_specs=pl.BlockSpec(memory_space=pltpu.SMEM),
    )(dots_pos.reshape(B // 128, 128), dots_neg.reshape(B * NEG // 128, 128))
    return out[0, 0]


# speedup vs baseline: 1.0283x; 1.0283x over previous
import functools

import jax
import jax.numpy as jnp
import numpy as np
from jax import lax
from jax.experimental import pallas as pl
from jax.experimental.pallas import tpu as pltpu
from jax.experimental.pallas import tpu_sc as plsc

B = 16384
V = 1000000
D = 64
NEG = 5

NC = 2
NS = 16
NW = NC * NS
L = 16

BPW = B // NW
C = 128
R = BPW // C
NV = D // L

TW = 2048
TBF = V // TW
TTAIL = V - TBF * TW
TB = TBF + 1
VP = TB * TW // 2
MAIN = TBF * TW
MP = MAIN // 2
T1 = MAIN + 512
P2 = MP + 256


def _transpose_block(in_hbm, tp_ref, out_ref, vin, sem):
    i = pl.program_id(0)

    @pl.when(i < TBF)
    def _main():
        cp = pltpu.make_async_copy(in_hbm.at[:, pl.ds(i * TW, TW)], vin, sem)
        cp.start()
        cp.wait()
        x = vin[...]
        eye = jnp.eye(128, dtype=jnp.float32)
        dn = (((1,), (1,)), ((), ()))
        for c in range(TW // 128):
            part = lax.dot_general(eye, x[:, 128 * c:128 * (c + 1)], dn)
            h, row = divmod(c, TW // 256)
            out_ref[pl.ds(128 * row, 128), pl.ds(h * D, D)] = part

    @pl.when(i == TBF)
    def _tail():
        cp = pltpu.make_async_copy(
            in_hbm.at[:, pl.ds(TBF * TW, 512)], vin.at[:, pl.ds(0, 512)], sem)
        cp.start()
        cp.wait()
        x = vin[:, pl.ds(0, 512)]
        eye = jnp.eye(128, dtype=jnp.float32)
        dn = (((1,), (1,)), ((), ()))
        for c in range(4):
            part = lax.dot_general(eye, x[:, 128 * c:128 * (c + 1)], dn)
            h, row = divmod(c, 2)
            out_ref[pl.ds(128 * row, 128), pl.ds(h * D, D)] = part
        out_ref[pl.ds(256, 32), :] = tp_ref[...]


def _to_pair_table(table_t, tail_pairs):
    return pl.pallas_call(
        _transpose_block,
        grid=(TB,),
        in_specs=[
            pl.BlockSpec(memory_space=pl.ANY),
            pl.BlockSpec((32, 2 * D), lambda i: (0, 0)),
        ],
        out_specs=pl.BlockSpec((TW // 2, 2 * D), lambda i: (i, 0)),
        out_shape=jax.ShapeDtypeStruct((VP, 2 * D), jnp.float32),
        scratch_shapes=[
            pltpu.VMEM((D, TW), jnp.float32),
            pltpu.SemaphoreType.DMA,
        ],
    )(table_t, tail_pairs)


@functools.partial(
    pl.kernel,
    out_type=[
        jax.ShapeDtypeStruct((B,), jnp.float32),
        jax.ShapeDtypeStruct((NEG * B,), jnp.float32),
    ],
    mesh=plsc.VectorSubcoreMesh(core_axis_name="c", subcore_axis_name="s"),
    compiler_params=pltpu.CompilerParams(needs_layout_passes=False),
    scratch_types=[
        pltpu.VMEM((C,), jnp.int32),
        pltpu.VMEM((C,), jnp.int32),
        pltpu.VMEM((C * NEG,), jnp.int32),
        pltpu.VMEM((C,), jnp.int32),
        pltpu.VMEM((C,), jnp.int32),
        pltpu.VMEM((C * NEG,), jnp.int32),
        pltpu.VMEM((C,), jnp.int32),
        pltpu.VMEM((C,), jnp.int32),
        pltpu.VMEM((C * NEG,), jnp.int32),
        pltpu.VMEM((C, 2 * D), jnp.float32),
        pltpu.VMEM((C, 2 * D), jnp.float32),
        pltpu.VMEM((C * NEG, 2 * D), jnp.float32),
        pltpu.VMEM((C,), jnp.float32),
        pltpu.VMEM((NEG, C), jnp.float32),
        pltpu.SemaphoreType.DMA,
    ],
)
def _sc_scores(pos_u, pos_v, neg_flat, u_pair, v_pair, out_pos, out_neg,
               idx_u, idx_v, idx_n, pair_u, pair_v, pair_n,
               off_u, off_v, off_n, u_rows, v_rows, n_rows,
               acc_p, acc_n, sem):
    wid = lax.axis_index("s") * NC + lax.axis_index("c")
    iota = lax.iota(jnp.int32, L)

    def round_body(r, _):
        base = wid * BPW + r * C
        pltpu.sync_copy(pos_u.at[pl.ds(base, C)], idx_u)
        pltpu.sync_copy(pos_v.at[pl.ds(base, C)], idx_v)
        pltpu.sync_copy(neg_flat.at[pl.ds(base * NEG, C * NEG)], idx_n)

        def pairsplit(i):
            pm = ((i >> 11) << 10) | (i & 1023)
            l = i - MAIN
            q = i - T1
            pp = jnp.where(i < MAIN, pm,
                           jnp.where(i < T1, MP + (l & 255), P2 + (q & 31)))
            ob = jnp.where(i < MAIN, (i >> 10) & 1,
                           jnp.where(i < T1, (l >> 8) & 1, (q >> 5) & 1))
            return pp, ob * D

        def shift_body(t, _):
            sl = pl.ds(t * L, L)
            pair_u[sl], off_u[sl] = pairsplit(idx_u[sl])
            pair_v[sl], off_v[sl] = pairsplit(idx_v[sl])
            return 0

        lax.fori_loop(0, C // L, shift_body, 0)

        def shift_n_body(t, _):
            sl = pl.ds(t * L, L)
            pair_n[sl], off_n[sl] = pairsplit(idx_n[sl])
            return 0

        lax.fori_loop(0, C * NEG // L, shift_n_body, 0)

        cu = pltpu.async_copy(u_pair.at[pair_u], u_rows, sem)
        cv = pltpu.async_copy(v_pair.at[pair_v], v_rows, sem)
        cn = pltpu.async_copy(v_pair.at[pair_n], n_rows, sem)
        cu.wait()
        cv.wait()
        cn.wait()

        def dot_bcast(u, w):
            p = u[0] * w[0]
            for kk in range(1, NV):
                p = p + u[kk] * w[kk]
            return jnp.full((L,), jnp.sum(p), jnp.float32)

        def load_row(ref, row, off):
            return [ref[row, pl.ds(off + L * kk, L)] for kk in range(NV)]

        def gbody(g, _):
            ou = off_u[pl.ds(g * L, L)]
            ov = off_v[pl.ds(g * L, L)]
            onn = [off_n[pl.ds(g * L * NEG + L * m, L)] for m in range(NEG)]
            accs = [jnp.zeros((L,), jnp.float32)] * (1 + NEG)
            for j in range(L):
                i = g * L + j
                lane = jnp.equal(iota, j)
                u = load_row(u_rows, i, ou[j])
                v = load_row(v_rows, i, ov[j])
                accs[0] = jnp.where(lane, dot_bcast(u, v), accs[0])
                for n in range(NEG):
                    k = j * NEG + n
                    w = load_row(n_rows, i * NEG + n, onn[k // L][k % L])
                    accs[1 + n] = jnp.where(lane, dot_bcast(u, w),
                                            accs[1 + n])
            acc_p[pl.ds(g * L, L)] = accs[0]
            for n in range(NEG):
                acc_n[n, pl.ds(g * L, L)] = accs[1 + n]
            return 0

        lax.fori_loop(0, C // L, gbody, 0)

        pltpu.sync_copy(acc_p, out_pos.at[pl.ds(base, C)])
        for n in range(NEG):
            pltpu.sync_copy(acc_n.at[n], out_neg.at[pl.ds(n * B + base, C)])
        return 0

    lax.fori_loop(0, R, round_body, 0)


def _tc_loss_kernel(p_ref, n_ref, o_ref):
    p = jnp.clip(p_ref[...], -10.0, 10.0)
    n = jnp.clip(n_ref[...], -10.0, 10.0)
    loss_pos = jnp.log1p(jnp.exp(-p))
    loss_neg = jnp.log1p(jnp.exp(n))
    o_ref[0, 0] = (jnp.sum(loss_pos) + jnp.sum(loss_neg)) * np.float32(1.0 / B)


def kernel(pos_u, pos_v, neg_v, u_table, v_table):
    pos_u = pos_u.astype(jnp.int32)
    pos_v = pos_v.astype(jnp.int32)
    neg_flat = neg_v.reshape(-1).astype(jnp.int32)

    ut = u_table[T1:]
    vt = v_table[T1:]
    u_pair = _to_pair_table(
        u_table.T, jnp.concatenate([ut[0:32], ut[32:64]], axis=1))
    v_pair = _to_pair_table(
        v_table.T, jnp.concatenate([vt[0:32], vt[32:64]], axis=1))

    dots_pos, dots_neg = _sc_scores(pos_u, pos_v, neg_flat, u_pair, v_pair)

    out = pl.pallas_call(
        _tc_loss_kernel,
        out_shape=jax.ShapeDtypeStruct((1, 1), jnp.float32),
        out_specs=pl.BlockSpec(memory_space=pltpu.SMEM),
    )(dots_pos.reshape(B // 128, 128), dots_neg.reshape(B * NEG // 128, 128))
    return out[0, 0]


# speedup vs baseline: 1.9394x; 1.8859x over previous
import functools

import jax
import jax.numpy as jnp
import numpy as np
from jax import lax
from jax.experimental import pallas as pl
from jax.experimental.pallas import tpu as pltpu
from jax.experimental.pallas import tpu_sc as plsc

B = 16384
V = 1000000
D = 64
NEG = 5

NC = 2
NS = 16
NW = NC * NS
L = 16

BPW = B // NW
C = 128
R = BPW // C
NV = D // L

TW = 2048
TBF = V // TW
TTAIL = V - TBF * TW
TB = TBF + 1
VP = TB * TW // 2
MAIN = TBF * TW
MP = MAIN // 2
T1 = MAIN + 512
P2 = MP + 256


def _main_copy(in_hbm, vin, sems, i, slot):
    return pltpu.make_async_copy(
        in_hbm.at[:, pl.ds(i * TW, TW)], vin.at[slot], sems.at[slot])


def _tail_copy(in_hbm, vin, sems, slot):
    return pltpu.make_async_copy(
        in_hbm.at[:, pl.ds(TBF * TW, 512)],
        vin.at[slot, :, pl.ds(0, 512)], sems.at[slot])


def _transpose_block(in_hbm, tp_ref, out_ref, vin, sems):
    i = pl.program_id(0)
    slot = lax.rem(i, 2)
    nxt = lax.rem(i + 1, 2)

    @pl.when(i == 0)
    def _prime():
        _main_copy(in_hbm, vin, sems, 0, 0).start()

    @pl.when(i + 1 < TBF)
    def _prefetch():
        _main_copy(in_hbm, vin, sems, i + 1, nxt).start()

    @pl.when(i + 1 == TBF)
    def _prefetch_tail():
        _tail_copy(in_hbm, vin, sems, nxt).start()

    eye = jnp.eye(128, dtype=jnp.float32)
    dn = (((1,), (1,)), ((), ()))

    @pl.when(i < TBF)
    def _main():
        _main_copy(in_hbm, vin, sems, i, slot).wait()
        x = vin[slot]
        for c in range(TW // 128):
            part = lax.dot_general(eye, x[:, 128 * c:128 * (c + 1)], dn)
            h, row = divmod(c, TW // 256)
            out_ref[pl.ds(128 * row, 128), pl.ds(h * D, D)] = part

    @pl.when(i == TBF)
    def _tail():
        _tail_copy(in_hbm, vin, sems, slot).wait()
        x = vin[slot, :, pl.ds(0, 512)]
        for c in range(4):
            part = lax.dot_general(eye, x[:, 128 * c:128 * (c + 1)], dn)
            h, row = divmod(c, 2)
            out_ref[pl.ds(128 * row, 128), pl.ds(h * D, D)] = part
        out_ref[pl.ds(256, 32), :] = tp_ref[...]


def _to_pair_table(table_t, tail_pairs):
    return pl.pallas_call(
        _transpose_block,
        grid=(TB,),
        in_specs=[
            pl.BlockSpec(memory_space=pl.ANY),
            pl.BlockSpec((32, 2 * D), lambda i: (0, 0)),
        ],
        out_specs=pl.BlockSpec((TW // 2, 2 * D), lambda i: (i, 0)),
        out_shape=jax.ShapeDtypeStruct((VP, 2 * D), jnp.float32),
        scratch_shapes=[
            pltpu.VMEM((2, D, TW), jnp.float32),
            pltpu.SemaphoreType.DMA((2,)),
        ],
    )(table_t, tail_pairs)


@functools.partial(
    pl.kernel,
    out_type=[
        jax.ShapeDtypeStruct((B,), jnp.float32),
        jax.ShapeDtypeStruct((NEG * B,), jnp.float32),
    ],
    mesh=plsc.VectorSubcoreMesh(core_axis_name="c", subcore_axis_name="s"),
    compiler_params=pltpu.CompilerParams(needs_layout_passes=False),
    scratch_types=[
        pltpu.VMEM((C,), jnp.int32),
        pltpu.VMEM((C,), jnp.int32),
        pltpu.VMEM((C * NEG,), jnp.int32),
        pltpu.VMEM((C,), jnp.int32),
        pltpu.VMEM((C,), jnp.int32),
        pltpu.VMEM((C * NEG,), jnp.int32),
        pltpu.VMEM((C,), jnp.int32),
        pltpu.VMEM((C,), jnp.int32),
        pltpu.VMEM((C * NEG,), jnp.int32),
        pltpu.VMEM((C, 2 * D), jnp.float32),
        pltpu.VMEM((C, 2 * D), jnp.float32),
        pltpu.VMEM((C * NEG, 2 * D), jnp.float32),
        pltpu.VMEM((C,), jnp.float32),
        pltpu.VMEM((NEG, C), jnp.float32),
        pltpu.SemaphoreType.DMA,
    ],
)
def _sc_scores(pos_u, pos_v, neg_flat, u_pair, v_pair, out_pos, out_neg,
               idx_u, idx_v, idx_n, pair_u, pair_v, pair_n,
               off_u, off_v, off_n, u_rows, v_rows, n_rows,
               acc_p, acc_n, sem):
    wid = lax.axis_index("s") * NC + lax.axis_index("c")
    iota = lax.iota(jnp.int32, L)

    def round_body(r, _):
        base = wid * BPW + r * C
        pltpu.sync_copy(pos_u.at[pl.ds(base, C)], idx_u)
        pltpu.sync_copy(pos_v.at[pl.ds(base, C)], idx_v)
        pltpu.sync_copy(neg_flat.at[pl.ds(base * NEG, C * NEG)], idx_n)

        def pairsplit(i):
            pm = ((i >> 11) << 10) | (i & 1023)
            l = i - MAIN
            q = i - T1
            pp = jnp.where(i < MAIN, pm,
                           jnp.where(i < T1, MP + (l & 255), P2 + (q & 31)))
            ob = jnp.where(i < MAIN, (i >> 10) & 1,
                           jnp.where(i < T1, (l >> 8) & 1, (q >> 5) & 1))
            return pp, ob * D

        def shift_body(t, _):
            sl = pl.ds(t * L, L)
            pair_u[sl], off_u[sl] = pairsplit(idx_u[sl])
            pair_v[sl], off_v[sl] = pairsplit(idx_v[sl])
            return 0

        lax.fori_loop(0, C // L, shift_body, 0)

        def shift_n_body(t, _):
            sl = pl.ds(t * L, L)
            pair_n[sl], off_n[sl] = pairsplit(idx_n[sl])
            return 0

        lax.fori_loop(0, C * NEG // L, shift_n_body, 0)

        cu = pltpu.async_copy(u_pair.at[pair_u], u_rows, sem)
        cv = pltpu.async_copy(v_pair.at[pair_v], v_rows, sem)
        cn = pltpu.async_copy(v_pair.at[pair_n], n_rows, sem)
        cu.wait()
        cv.wait()
        cn.wait()

        def dot_bcast(u, w):
            p = u[0] * w[0]
            for kk in range(1, NV):
                p = p + u[kk] * w[kk]
            return jnp.full((L,), jnp.sum(p), jnp.float32)

        def load_row(ref, row, off):
            return [ref[row, pl.ds(off + L * kk, L)] for kk in range(NV)]

        def gbody(g, _):
            ou = off_u[pl.ds(g * L, L)]
            ov = off_v[pl.ds(g * L, L)]
            onn = [off_n[pl.ds(g * L * NEG + L * m, L)] for m in range(NEG)]
            accs = [jnp.zeros((L,), jnp.float32)] * (1 + NEG)
            for j in range(L):
                i = g * L + j
                lane = jnp.equal(iota, j)
                u = load_row(u_rows, i, ou[j])
                v = load_row(v_rows, i, ov[j])
                accs[0] = jnp.where(lane, dot_bcast(u, v), accs[0])
                for n in range(NEG):
                    k = j * NEG + n
                    w = load_row(n_rows, i * NEG + n, onn[k // L][k % L])
                    accs[1 + n] = jnp.where(lane, dot_bcast(u, w),
                                            accs[1 + n])
            acc_p[pl.ds(g * L, L)] = accs[0]
            for n in range(NEG):
                acc_n[n, pl.ds(g * L, L)] = accs[1 + n]
            return 0

        lax.fori_loop(0, C // L, gbody, 0)

        pltpu.sync_copy(acc_p, out_pos.at[pl.ds(base, C)])
        for n in range(NEG):
            pltpu.sync_copy(acc_n.at[n], out_neg.at[pl.ds(n * B + base, C)])
        return 0

    lax.fori_loop(0, R, round_body, 0)


def _tc_loss_kernel(p_ref, n_ref, o_ref):
    p = jnp.clip(p_ref[...], -10.0, 10.0)
    n = jnp.clip(n_ref[...], -10.0, 10.0)
    loss_pos = jnp.log1p(jnp.exp(-p))
    loss_neg = jnp.log1p(jnp.exp(n))
    o_ref[0, 0] = (jnp.sum(loss_pos) + jnp.sum(loss_neg)) * np.float32(1.0 / B)


def kernel(pos_u, pos_v, neg_v, u_table, v_table):
    pos_u = pos_u.astype(jnp.int32)
    pos_v = pos_v.astype(jnp.int32)
    neg_flat = neg_v.reshape(-1).astype(jnp.int32)

    ut = u_table[T1:]
    vt = v_table[T1:]
    u_pair = _to_pair_table(
        u_table.T, jnp.concatenate([ut[0:32], ut[32:64]], axis=1))
    v_pair = _to_pair_table(
        v_table.T, jnp.concatenate([vt[0:32], vt[32:64]], axis=1))

    dots_pos, dots_neg = _sc_scores(pos_u, pos_v, neg_flat, u_pair, v_pair)

    out = pl.pallas_call(
        _tc_loss_kernel,
        out_shape=jax.ShapeDtypeStruct((1, 1), jnp.float32),
        out_specs=pl.BlockSpec(memory_space=pltpu.SMEM),
    )(dots_pos.reshape(B // 128, 128), dots_neg.reshape(B * NEG // 128, 128))
    return out[0, 0]


# speedup vs baseline: 2.6110x; 1.3463x over previous
import functools

import jax
import jax.numpy as jnp
import numpy as np
from jax import lax
from jax.experimental import pallas as pl
from jax.experimental.pallas import tpu as pltpu
from jax.experimental.pallas import tpu_sc as plsc

B = 16384
V = 1000000
D = 64
NEG = 5

NC = 2
NS = 16
NW = NC * NS
L = 16

BPW = B // NW
C = 128
R = BPW // C
NV = D // L

TW = 4096
TBF = V // TW
TTAIL = V - TBF * TW
TB = TBF + 1
VP = TB * TW // 2
MAIN = TBF * TW
MP = MAIN // 2
T1 = MAIN + 512
P2 = MP + 256


def _main_copy(in_hbm, vin, sems, i, slot):
    return pltpu.make_async_copy(
        in_hbm.at[:, pl.ds(i * TW, TW)], vin.at[slot], sems.at[slot])


def _tail_copy(in_hbm, vin, sems, slot):
    return pltpu.make_async_copy(
        in_hbm.at[:, pl.ds(TBF * TW, 512)],
        vin.at[slot, :, pl.ds(0, 512)], sems.at[slot])


def _transpose_block(in_hbm, tp_ref, out_ref, vin, sems):
    i = pl.program_id(0)
    slot = lax.rem(i, 2)
    nxt = lax.rem(i + 1, 2)

    @pl.when(i == 0)
    def _prime():
        _main_copy(in_hbm, vin, sems, 0, 0).start()

    @pl.when(i + 1 < TBF)
    def _prefetch():
        _main_copy(in_hbm, vin, sems, i + 1, nxt).start()

    @pl.when(i + 1 == TBF)
    def _prefetch_tail():
        _tail_copy(in_hbm, vin, sems, nxt).start()

    CH = 256
    eye = jnp.eye(CH, dtype=jnp.float32)
    dn = (((1,), (1,)), ((), ()))

    @pl.when(i < TBF)
    def _main():
        _main_copy(in_hbm, vin, sems, i, slot).wait()
        x = vin[slot]
        for c in range(TW // CH):
            part = lax.dot_general(eye, x[:, CH * c:CH * (c + 1)], dn)
            h, row = divmod(c, TW // (2 * CH))
            out_ref[pl.ds(CH * row, CH), pl.ds(h * D, D)] = part

    @pl.when(i == TBF)
    def _tail():
        _tail_copy(in_hbm, vin, sems, slot).wait()
        x = vin[slot, :, pl.ds(0, 512)]
        for c in range(512 // CH):
            part = lax.dot_general(eye, x[:, CH * c:CH * (c + 1)], dn)
            h, row = divmod(c, 512 // (2 * CH))
            out_ref[pl.ds(CH * row, CH), pl.ds(h * D, D)] = part
        out_ref[pl.ds(256, 32), :] = tp_ref[...]


def _to_pair_table(table_t, tail_pairs):
    return pl.pallas_call(
        _transpose_block,
        grid=(TB,),
        in_specs=[
            pl.BlockSpec(memory_space=pl.ANY),
            pl.BlockSpec((32, 2 * D), lambda i: (0, 0)),
        ],
        out_specs=pl.BlockSpec((TW // 2, 2 * D), lambda i: (i, 0)),
        out_shape=jax.ShapeDtypeStruct((VP, 2 * D), jnp.float32),
        scratch_shapes=[
            pltpu.VMEM((2, D, TW), jnp.float32),
            pltpu.SemaphoreType.DMA((2,)),
        ],
    )(table_t, tail_pairs)


@functools.partial(
    pl.kernel,
    out_type=[
        jax.ShapeDtypeStruct((B,), jnp.float32),
        jax.ShapeDtypeStruct((NEG * B,), jnp.float32),
    ],
    mesh=plsc.VectorSubcoreMesh(core_axis_name="c", subcore_axis_name="s"),
    compiler_params=pltpu.CompilerParams(needs_layout_passes=False),
    scratch_types=[
        pltpu.VMEM((C,), jnp.int32),
        pltpu.VMEM((C,), jnp.int32),
        pltpu.VMEM((C * NEG,), jnp.int32),
        pltpu.VMEM((C,), jnp.int32),
        pltpu.VMEM((C,), jnp.int32),
        pltpu.VMEM((C * NEG,), jnp.int32),
        pltpu.VMEM((C,), jnp.int32),
        pltpu.VMEM((C,), jnp.int32),
        pltpu.VMEM((C * NEG,), jnp.int32),
        pltpu.VMEM((C, 2 * D), jnp.float32),
        pltpu.VMEM((C, 2 * D), jnp.float32),
        pltpu.VMEM((C * NEG, 2 * D), jnp.float32),
        pltpu.VMEM((C,), jnp.float32),
        pltpu.VMEM((NEG, C), jnp.float32),
        pltpu.SemaphoreType.DMA,
    ],
)
def _sc_scores(pos_u, pos_v, neg_flat, u_pair, v_pair, out_pos, out_neg,
               idx_u, idx_v, idx_n, pair_u, pair_v, pair_n,
               off_u, off_v, off_n, u_rows, v_rows, n_rows,
               acc_p, acc_n, sem):
    wid = lax.axis_index("s") * NC + lax.axis_index("c")
    iota = lax.iota(jnp.int32, L)

    def round_body(r, _):
        base = wid * BPW + r * C
        pltpu.sync_copy(pos_u.at[pl.ds(base, C)], idx_u)
        pltpu.sync_copy(pos_v.at[pl.ds(base, C)], idx_v)
        pltpu.sync_copy(neg_flat.at[pl.ds(base * NEG, C * NEG)], idx_n)

        def pairsplit(i):
            pm = ((i >> 12) << 11) | (i & 2047)
            l = i - MAIN
            q = i - T1
            pp = jnp.where(i < MAIN, pm,
                           jnp.where(i < T1, MP + (l & 255), P2 + (q & 31)))
            ob = jnp.where(i < MAIN, (i >> 11) & 1,
                           jnp.where(i < T1, (l >> 8) & 1, (q >> 5) & 1))
            return pp, ob * D

        def shift_body(t, _):
            sl = pl.ds(t * L, L)
            pair_u[sl], off_u[sl] = pairsplit(idx_u[sl])
            pair_v[sl], off_v[sl] = pairsplit(idx_v[sl])
            return 0

        lax.fori_loop(0, C // L, shift_body, 0)

        def shift_n_body(t, _):
            sl = pl.ds(t * L, L)
            pair_n[sl], off_n[sl] = pairsplit(idx_n[sl])
            return 0

        lax.fori_loop(0, C * NEG // L, shift_n_body, 0)

        cu = pltpu.async_copy(u_pair.at[pair_u], u_rows, sem)
        cv = pltpu.async_copy(v_pair.at[pair_v], v_rows, sem)
        cn = pltpu.async_copy(v_pair.at[pair_n], n_rows, sem)
        cu.wait()
        cv.wait()
        cn.wait()

        def dot_bcast(u, w):
            p = u[0] * w[0]
            for kk in range(1, NV):
                p = p + u[kk] * w[kk]
            return jnp.full((L,), jnp.sum(p), jnp.float32)

        def load_row(ref, row, off):
            return [ref[row, pl.ds(off + L * kk, L)] for kk in range(NV)]

        def gbody(g, _):
            ou = off_u[pl.ds(g * L, L)]
            ov = off_v[pl.ds(g * L, L)]
            onn = [off_n[pl.ds(g * L * NEG + L * m, L)] for m in range(NEG)]
            accs = [jnp.zeros((L,), jnp.float32)] * (1 + NEG)
            for j in range(L):
                i = g * L + j
                lane = jnp.equal(iota, j)
                u = load_row(u_rows, i, ou[j])
                v = load_row(v_rows, i, ov[j])
                accs[0] = jnp.where(lane, dot_bcast(u, v), accs[0])
                for n in range(NEG):
                    k = j * NEG + n
                    w = load_row(n_rows, i * NEG + n, onn[k // L][k % L])
                    accs[1 + n] = jnp.where(lane, dot_bcast(u, w),
                                            accs[1 + n])
            acc_p[pl.ds(g * L, L)] = accs[0]
            for n in range(NEG):
                acc_n[n, pl.ds(g * L, L)] = accs[1 + n]
            return 0

        lax.fori_loop(0, C // L, gbody, 0)

        pltpu.sync_copy(acc_p, out_pos.at[pl.ds(base, C)])
        for n in range(NEG):
            pltpu.sync_copy(acc_n.at[n], out_neg.at[pl.ds(n * B + base, C)])
        return 0

    lax.fori_loop(0, R, round_body, 0)


def _tc_loss_kernel(p_ref, n_ref, o_ref):
    p = jnp.clip(p_ref[...], -10.0, 10.0)
    n = jnp.clip(n_ref[...], -10.0, 10.0)
    loss_pos = jnp.log1p(jnp.exp(-p))
    loss_neg = jnp.log1p(jnp.exp(n))
    o_ref[0, 0] = (jnp.sum(loss_pos) + jnp.sum(loss_neg)) * np.float32(1.0 / B)


def kernel(pos_u, pos_v, neg_v, u_table, v_table):
    pos_u = pos_u.astype(jnp.int32)
    pos_v = pos_v.astype(jnp.int32)
    neg_flat = neg_v.reshape(-1).astype(jnp.int32)

    ut = u_table[T1:]
    vt = v_table[T1:]
    u_pair = _to_pair_table(
        u_table.T, jnp.concatenate([ut[0:32], ut[32:64]], axis=1))
    v_pair = _to_pair_table(
        v_table.T, jnp.concatenate([vt[0:32], vt[32:64]], axis=1))

    dots_pos, dots_neg = _sc_scores(pos_u, pos_v, neg_flat, u_pair, v_pair)

    out = pl.pallas_call(
        _tc_loss_kernel,
        out_shape=jax.ShapeDtypeStruct((1, 1), jnp.float32),
        out_specs=pl.BlockSpec(memory_space=pltpu.SMEM),
    )(dots_pos.reshape(B // 128, 128), dots_neg.reshape(B * NEG // 128, 128))
    return out[0, 0]


# speedup vs baseline: 2.6951x; 1.0322x over previous
import functools

import jax
import jax.numpy as jnp
import numpy as np
from jax import lax
from jax.experimental import pallas as pl
from jax.experimental.pallas import tpu as pltpu
from jax.experimental.pallas import tpu_sc as plsc

B = 16384
V = 1000000
D = 64
NEG = 5

NC = 2
NS = 16
NW = NC * NS
L = 16

BPW = B // NW
C = 128
R = BPW // C
NV = D // L

TW = 4096
TBF = V // TW
TTAIL = V - TBF * TW
TB = TBF + 1
VP = TB * TW // 2
MAIN = TBF * TW
MP = MAIN // 2
T1 = MAIN + 512
P2 = MP + 256


def _main_copy(in_hbm, vin, sems, i, slot):
    return pltpu.make_async_copy(
        in_hbm.at[:, pl.ds(i * TW, TW)], vin.at[slot], sems.at[slot])


def _tail_copy(in_hbm, vin, sems, slot):
    return pltpu.make_async_copy(
        in_hbm.at[:, pl.ds(TBF * TW, 512)],
        vin.at[slot, :, pl.ds(0, 512)], sems.at[slot])


def _transpose_block(in_hbm, tp_ref, out_ref, vin, sems):
    i = pl.program_id(0)
    slot = lax.rem(i, 2)
    nxt = lax.rem(i + 1, 2)

    @pl.when(i == 0)
    def _prime():
        _main_copy(in_hbm, vin, sems, 0, 0).start()

    @pl.when(i + 1 < TBF)
    def _prefetch():
        _main_copy(in_hbm, vin, sems, i + 1, nxt).start()

    @pl.when(i + 1 == TBF)
    def _prefetch_tail():
        _tail_copy(in_hbm, vin, sems, nxt).start()

    CH = 128
    eye = jnp.eye(CH, dtype=jnp.float32)
    dn = (((1,), (1,)), ((), ()))

    @pl.when(i < TBF)
    def _main():
        _main_copy(in_hbm, vin, sems, i, slot).wait()
        x = vin[slot]
        for c in range(TW // CH):
            part = lax.dot_general(eye, x[:, CH * c:CH * (c + 1)], dn)
            h, row = divmod(c, TW // (2 * CH))
            out_ref[pl.ds(CH * row, CH), pl.ds(h * D, D)] = part

    @pl.when(i == TBF)
    def _tail():
        _tail_copy(in_hbm, vin, sems, slot).wait()
        x = vin[slot, :, pl.ds(0, 512)]
        for c in range(512 // CH):
            part = lax.dot_general(eye, x[:, CH * c:CH * (c + 1)], dn)
            h, row = divmod(c, 512 // (2 * CH))
            out_ref[pl.ds(CH * row, CH), pl.ds(h * D, D)] = part
        out_ref[pl.ds(256, 32), :] = tp_ref[...]


def _to_pair_table(table_t, tail_pairs):
    return pl.pallas_call(
        _transpose_block,
        grid=(TB,),
        in_specs=[
            pl.BlockSpec(memory_space=pl.ANY),
            pl.BlockSpec((32, 2 * D), lambda i: (0, 0)),
        ],
        out_specs=pl.BlockSpec((TW // 2, 2 * D), lambda i: (i, 0)),
        out_shape=jax.ShapeDtypeStruct((VP, 2 * D), jnp.float32),
        scratch_shapes=[
            pltpu.VMEM((2, D, TW), jnp.float32),
            pltpu.SemaphoreType.DMA((2,)),
        ],
    )(table_t, tail_pairs)


@functools.partial(
    pl.kernel,
    out_type=[
        jax.ShapeDtypeStruct((B,), jnp.float32),
        jax.ShapeDtypeStruct((NEG * B,), jnp.float32),
    ],
    mesh=plsc.VectorSubcoreMesh(core_axis_name="c", subcore_axis_name="s"),
    compiler_params=pltpu.CompilerParams(needs_layout_passes=False),
    scratch_types=[
        pltpu.VMEM((C,), jnp.int32),
        pltpu.VMEM((C,), jnp.int32),
        pltpu.VMEM((C * NEG,), jnp.int32),
        pltpu.VMEM((C,), jnp.int32),
        pltpu.VMEM((C,), jnp.int32),
        pltpu.VMEM((C * NEG,), jnp.int32),
        pltpu.VMEM((C,), jnp.int32),
        pltpu.VMEM((C,), jnp.int32),
        pltpu.VMEM((C * NEG,), jnp.int32),
        pltpu.VMEM((C, 2 * D), jnp.float32),
        pltpu.VMEM((C, 2 * D), jnp.float32),
        pltpu.VMEM((C * NEG, 2 * D), jnp.float32),
        pltpu.VMEM((C,), jnp.float32),
        pltpu.VMEM((NEG, C), jnp.float32),
        pltpu.SemaphoreType.DMA,
    ],
)
def _sc_scores(pos_u, pos_v, neg_flat, u_pair, v_pair, out_pos, out_neg,
               idx_u, idx_v, idx_n, pair_u, pair_v, pair_n,
               off_u, off_v, off_n, u_rows, v_rows, n_rows,
               acc_p, acc_n, sem):
    wid = lax.axis_index("s") * NC + lax.axis_index("c")
    iota = lax.iota(jnp.int32, L)

    def round_body(r, _):
        base = wid * BPW + r * C
        pltpu.sync_copy(pos_u.at[pl.ds(base, C)], idx_u)
        pltpu.sync_copy(pos_v.at[pl.ds(base, C)], idx_v)
        pltpu.sync_copy(neg_flat.at[pl.ds(base * NEG, C * NEG)], idx_n)

        def pairsplit(i):
            pm = ((i >> 12) << 11) | (i & 2047)
            l = i - MAIN
            q = i - T1
            pp = jnp.where(i < MAIN, pm,
                           jnp.where(i < T1, MP + (l & 255), P2 + (q & 31)))
            ob = jnp.where(i < MAIN, (i >> 11) & 1,
                           jnp.where(i < T1, (l >> 8) & 1, (q >> 5) & 1))
            return pp, ob * D

        def shift_body(t, _):
            sl = pl.ds(t * L, L)
            pair_u[sl], off_u[sl] = pairsplit(idx_u[sl])
            pair_v[sl], off_v[sl] = pairsplit(idx_v[sl])
            return 0

        lax.fori_loop(0, C // L, shift_body, 0)

        def shift_n_body(t, _):
            sl = pl.ds(t * L, L)
            pair_n[sl], off_n[sl] = pairsplit(idx_n[sl])
            return 0

        lax.fori_loop(0, C * NEG // L, shift_n_body, 0)

        cu = pltpu.async_copy(u_pair.at[pair_u], u_rows, sem)
        cv = pltpu.async_copy(v_pair.at[pair_v], v_rows, sem)
        cn = pltpu.async_copy(v_pair.at[pair_n], n_rows, sem)
        cu.wait()
        cv.wait()
        cn.wait()

        def dot_bcast(u, w):
            p = u[0] * w[0]
            for kk in range(1, NV):
                p = p + u[kk] * w[kk]
            return jnp.full((L,), jnp.sum(p), jnp.float32)

        def load_row(ref, row, off):
            return [ref[row, pl.ds(off + L * kk, L)] for kk in range(NV)]

        def gbody(g, _):
            ou = off_u[pl.ds(g * L, L)]
            ov = off_v[pl.ds(g * L, L)]
            onn = [off_n[pl.ds(g * L * NEG + L * m, L)] for m in range(NEG)]
            accs = [jnp.zeros((L,), jnp.float32)] * (1 + NEG)
            for j in range(L):
                i = g * L + j
                lane = jnp.equal(iota, j)
                u = load_row(u_rows, i, ou[j])
                v = load_row(v_rows, i, ov[j])
                accs[0] = jnp.where(lane, dot_bcast(u, v), accs[0])
                for n in range(NEG):
                    k = j * NEG + n
                    w = load_row(n_rows, i * NEG + n, onn[k // L][k % L])
                    accs[1 + n] = jnp.where(lane, dot_bcast(u, w),
                                            accs[1 + n])
            acc_p[pl.ds(g * L, L)] = accs[0]
            for n in range(NEG):
                acc_n[n, pl.ds(g * L, L)] = accs[1 + n]
            return 0

        lax.fori_loop(0, C // L, gbody, 0)

        pltpu.sync_copy(acc_p, out_pos.at[pl.ds(base, C)])
        for n in range(NEG):
            pltpu.sync_copy(acc_n.at[n], out_neg.at[pl.ds(n * B + base, C)])
        return 0

    lax.fori_loop(0, R, round_body, 0)


def _tc_loss_kernel(p_ref, n_ref, o_ref):
    p = jnp.clip(p_ref[...], -10.0, 10.0)
    n = jnp.clip(n_ref[...], -10.0, 10.0)
    loss_pos = jnp.log1p(jnp.exp(-p))
    loss_neg = jnp.log1p(jnp.exp(n))
    o_ref[0, 0] = (jnp.sum(loss_pos) + jnp.sum(loss_neg)) * np.float32(1.0 / B)


def kernel(pos_u, pos_v, neg_v, u_table, v_table):
    pos_u = pos_u.astype(jnp.int32)
    pos_v = pos_v.astype(jnp.int32)
    neg_flat = neg_v.reshape(-1).astype(jnp.int32)

    ut = u_table[T1:]
    vt = v_table[T1:]
    u_pair = _to_pair_table(
        u_table.T, jnp.concatenate([ut[0:32], ut[32:64]], axis=1))
    v_pair = _to_pair_table(
        v_table.T, jnp.concatenate([vt[0:32], vt[32:64]], axis=1))

    dots_pos, dots_neg = _sc_scores(pos_u, pos_v, neg_flat, u_pair, v_pair)

    out = pl.pallas_call(
        _tc_loss_kernel,
        out_shape=jax.ShapeDtypeStruct((1, 1), jnp.float32),
        out_specs=pl.BlockSpec(memory_space=pltpu.SMEM),
    )(dots_pos.reshape(B // 128, 128), dots_neg.reshape(B * NEG // 128, 128))
    return out[0, 0]


# speedup vs baseline: 3.4261x; 1.2713x over previous
import functools

import jax
import jax.numpy as jnp
import numpy as np
from jax import lax
from jax.experimental import pallas as pl
from jax.experimental.pallas import tpu as pltpu
from jax.experimental.pallas import tpu_sc as plsc

B = 16384
V = 1000000
D = 64
NEG = 5

NC = 2
NS = 16
NW = NC * NS
L = 16

BPW = B // NW
C = 128
R = BPW // C
NV = D // L

TW = 8192
TBF = V // TW
TTAIL = V - TBF * TW
TB = TBF + 1
VP = TB * TW // 2
MAIN = TBF * TW
MP = MAIN // 2
T1 = MAIN + 512
P2 = MP + 256


def _main_copy(in_hbm, vin, sems, i, slot):
    return pltpu.make_async_copy(
        in_hbm.at[:, pl.ds(i * TW, TW)], vin.at[slot], sems.at[slot])


def _tail_copy(in_hbm, vin, sems, slot):
    return pltpu.make_async_copy(
        in_hbm.at[:, pl.ds(TBF * TW, 512)],
        vin.at[slot, :, pl.ds(0, 512)], sems.at[slot])


def _transpose_block(in_hbm, tp_ref, out_ref, vin, sems):
    i = pl.program_id(0)
    slot = lax.rem(i, 2)
    nxt = lax.rem(i + 1, 2)

    @pl.when(i == 0)
    def _prime():
        _main_copy(in_hbm, vin, sems, 0, 0).start()

    @pl.when(i + 1 < TBF)
    def _prefetch():
        _main_copy(in_hbm, vin, sems, i + 1, nxt).start()

    @pl.when(i + 1 == TBF)
    def _prefetch_tail():
        _tail_copy(in_hbm, vin, sems, nxt).start()

    CH = 128
    eye = jnp.eye(CH, dtype=jnp.float32)
    dn = (((1,), (1,)), ((), ()))

    @pl.when(i < TBF)
    def _main():
        _main_copy(in_hbm, vin, sems, i, slot).wait()
        x = vin[slot]
        for c in range(TW // CH):
            part = lax.dot_general(eye, x[:, CH * c:CH * (c + 1)], dn)
            h, row = divmod(c, TW // (2 * CH))
            out_ref[pl.ds(CH * row, CH), pl.ds(h * D, D)] = part

    @pl.when(i == TBF)
    def _tail():
        _tail_copy(in_hbm, vin, sems, slot).wait()
        x = vin[slot, :, pl.ds(0, 512)]
        for c in range(512 // CH):
            part = lax.dot_general(eye, x[:, CH * c:CH * (c + 1)], dn)
            h, row = divmod(c, 512 // (2 * CH))
            out_ref[pl.ds(CH * row, CH), pl.ds(h * D, D)] = part
        out_ref[pl.ds(256, 32), :] = tp_ref[...]


def _to_pair_table(table_t, tail_pairs):
    return pl.pallas_call(
        _transpose_block,
        grid=(TB,),
        in_specs=[
            pl.BlockSpec(memory_space=pl.ANY),
            pl.BlockSpec((32, 2 * D), lambda i: (0, 0)),
        ],
        out_specs=pl.BlockSpec((TW // 2, 2 * D), lambda i: (i, 0)),
        out_shape=jax.ShapeDtypeStruct((VP, 2 * D), jnp.float32),
        scratch_shapes=[
            pltpu.VMEM((2, D, TW), jnp.float32),
            pltpu.SemaphoreType.DMA((2,)),
        ],
    )(table_t, tail_pairs)


@functools.partial(
    pl.kernel,
    out_type=[
        jax.ShapeDtypeStruct((B,), jnp.float32),
        jax.ShapeDtypeStruct((NEG * B,), jnp.float32),
    ],
    mesh=plsc.VectorSubcoreMesh(core_axis_name="c", subcore_axis_name="s"),
    compiler_params=pltpu.CompilerParams(needs_layout_passes=False),
    scratch_types=[
        pltpu.VMEM((C,), jnp.int32),
        pltpu.VMEM((C,), jnp.int32),
        pltpu.VMEM((C * NEG,), jnp.int32),
        pltpu.VMEM((C,), jnp.int32),
        pltpu.VMEM((C,), jnp.int32),
        pltpu.VMEM((C * NEG,), jnp.int32),
        pltpu.VMEM((C,), jnp.int32),
        pltpu.VMEM((C,), jnp.int32),
        pltpu.VMEM((C * NEG,), jnp.int32),
        pltpu.VMEM((C, 2 * D), jnp.float32),
        pltpu.VMEM((C, 2 * D), jnp.float32),
        pltpu.VMEM((C * NEG, 2 * D), jnp.float32),
        pltpu.VMEM((C,), jnp.float32),
        pltpu.VMEM((NEG, C), jnp.float32),
        pltpu.SemaphoreType.DMA,
    ],
)
def _sc_scores(pos_u, pos_v, neg_flat, u_pair, v_pair, out_pos, out_neg,
               idx_u, idx_v, idx_n, pair_u, pair_v, pair_n,
               off_u, off_v, off_n, u_rows, v_rows, n_rows,
               acc_p, acc_n, sem):
    wid = lax.axis_index("s") * NC + lax.axis_index("c")
    iota = lax.iota(jnp.int32, L)

    def round_body(r, _):
        base = wid * BPW + r * C
        pltpu.sync_copy(pos_u.at[pl.ds(base, C)], idx_u)
        pltpu.sync_copy(pos_v.at[pl.ds(base, C)], idx_v)
        pltpu.sync_copy(neg_flat.at[pl.ds(base * NEG, C * NEG)], idx_n)

        def pairsplit(i):
            pm = ((i >> 13) << 12) | (i & 4095)
            l = i - MAIN
            q = i - T1
            pp = jnp.where(i < MAIN, pm,
                           jnp.where(i < T1, MP + (l & 255), P2 + (q & 31)))
            ob = jnp.where(i < MAIN, (i >> 12) & 1,
                           jnp.where(i < T1, (l >> 8) & 1, (q >> 5) & 1))
            return pp, ob * D

        def shift_body(t, _):
            sl = pl.ds(t * L, L)
            pair_u[sl], off_u[sl] = pairsplit(idx_u[sl])
            pair_v[sl], off_v[sl] = pairsplit(idx_v[sl])
            return 0

        lax.fori_loop(0, C // L, shift_body, 0)

        def shift_n_body(t, _):
            sl = pl.ds(t * L, L)
            pair_n[sl], off_n[sl] = pairsplit(idx_n[sl])
            return 0

        lax.fori_loop(0, C * NEG // L, shift_n_body, 0)

        cu = pltpu.async_copy(u_pair.at[pair_u], u_rows, sem)
        cv = pltpu.async_copy(v_pair.at[pair_v], v_rows, sem)
        cn = pltpu.async_copy(v_pair.at[pair_n], n_rows, sem)
        cu.wait()
        cv.wait()
        cn.wait()

        def dot_bcast(u, w):
            p = u[0] * w[0]
            for kk in range(1, NV):
                p = p + u[kk] * w[kk]
            return jnp.full((L,), jnp.sum(p), jnp.float32)

        def load_row(ref, row, off):
            return [ref[row, pl.ds(off + L * kk, L)] for kk in range(NV)]

        def gbody(g, _):
            ou = off_u[pl.ds(g * L, L)]
            ov = off_v[pl.ds(g * L, L)]
            onn = [off_n[pl.ds(g * L * NEG + L * m, L)] for m in range(NEG)]
            accs = [jnp.zeros((L,), jnp.float32)] * (1 + NEG)
            for j in range(L):
                i = g * L + j
                lane = jnp.equal(iota, j)
                u = load_row(u_rows, i, ou[j])
                v = load_row(v_rows, i, ov[j])
                accs[0] = jnp.where(lane, dot_bcast(u, v), accs[0])
                for n in range(NEG):
                    k = j * NEG + n
                    w = load_row(n_rows, i * NEG + n, onn[k // L][k % L])
                    accs[1 + n] = jnp.where(lane, dot_bcast(u, w),
                                            accs[1 + n])
            acc_p[pl.ds(g * L, L)] = accs[0]
            for n in range(NEG):
                acc_n[n, pl.ds(g * L, L)] = accs[1 + n]
            return 0

        lax.fori_loop(0, C // L, gbody, 0)

        pltpu.sync_copy(acc_p, out_pos.at[pl.ds(base, C)])
        for n in range(NEG):
            pltpu.sync_copy(acc_n.at[n], out_neg.at[pl.ds(n * B + base, C)])
        return 0

    lax.fori_loop(0, R, round_body, 0)


def _tc_loss_kernel(p_ref, n_ref, o_ref):
    p = jnp.clip(p_ref[...], -10.0, 10.0)
    n = jnp.clip(n_ref[...], -10.0, 10.0)
    loss_pos = jnp.log1p(jnp.exp(-p))
    loss_neg = jnp.log1p(jnp.exp(n))
    o_ref[0, 0] = (jnp.sum(loss_pos) + jnp.sum(loss_neg)) * np.float32(1.0 / B)


def kernel(pos_u, pos_v, neg_v, u_table, v_table):
    pos_u = pos_u.astype(jnp.int32)
    pos_v = pos_v.astype(jnp.int32)
    neg_flat = neg_v.reshape(-1).astype(jnp.int32)

    ut = u_table[T1:]
    vt = v_table[T1:]
    u_pair = _to_pair_table(
        u_table.T, jnp.concatenate([ut[0:32], ut[32:64]], axis=1))
    v_pair = _to_pair_table(
        v_table.T, jnp.concatenate([vt[0:32], vt[32:64]], axis=1))

    dots_pos, dots_neg = _sc_scores(pos_u, pos_v, neg_flat, u_pair, v_pair)

    out = pl.pallas_call(
        _tc_loss_kernel,
        out_shape=jax.ShapeDtypeStruct((1, 1), jnp.float32),
        out_specs=pl.BlockSpec(memory_space=pltpu.SMEM),
    )(dots_pos.reshape(B // 128, 128), dots_neg.reshape(B * NEG // 128, 128))
    return out[0, 0]


# speedup vs baseline: 3.9787x; 1.1613x over previous
import functools

import jax
import jax.numpy as jnp
import numpy as np
from jax import lax
from jax.experimental import pallas as pl
from jax.experimental.pallas import tpu as pltpu
from jax.experimental.pallas import tpu_sc as plsc

B = 16384
V = 1000000
D = 64
NEG = 5

NC = 2
NS = 16
NW = NC * NS
L = 16

BPW = B // NW
C = 128
R = BPW // C
NV = D // L

TW = 16384
TBF = V // TW
TTAIL = V - TBF * TW
TB = TBF + 1
VP = TB * TW // 2
MAIN = TBF * TW
MP = MAIN // 2
T1 = MAIN + 512
P2 = MP + 256


def _main_copy(in_hbm, vin, sems, i, slot):
    return pltpu.make_async_copy(
        in_hbm.at[:, pl.ds(i * TW, TW)], vin.at[slot], sems.at[slot])


def _tail_copy(in_hbm, vin, sems, slot):
    return pltpu.make_async_copy(
        in_hbm.at[:, pl.ds(TBF * TW, 512)],
        vin.at[slot, :, pl.ds(0, 512)], sems.at[slot])


def _transpose_block(in_hbm, tp_ref, out_ref, vin, sems):
    i = pl.program_id(0)
    slot = lax.rem(i, 2)
    nxt = lax.rem(i + 1, 2)

    @pl.when(i == 0)
    def _prime():
        _main_copy(in_hbm, vin, sems, 0, 0).start()

    @pl.when(i + 1 < TBF)
    def _prefetch():
        _main_copy(in_hbm, vin, sems, i + 1, nxt).start()

    @pl.when(i + 1 == TBF)
    def _prefetch_tail():
        _tail_copy(in_hbm, vin, sems, nxt).start()

    CH = 128
    eye = jnp.eye(CH, dtype=jnp.float32)
    dn = (((1,), (1,)), ((), ()))

    @pl.when(i < TBF)
    def _main():
        _main_copy(in_hbm, vin, sems, i, slot).wait()
        x = vin[slot]
        for c in range(TW // CH):
            part = lax.dot_general(eye, x[:, CH * c:CH * (c + 1)], dn)
            h, row = divmod(c, TW // (2 * CH))
            out_ref[pl.ds(CH * row, CH), pl.ds(h * D, D)] = part

    @pl.when(i == TBF)
    def _tail():
        _tail_copy(in_hbm, vin, sems, slot).wait()
        x = vin[slot, :, pl.ds(0, 512)]
        for c in range(512 // CH):
            part = lax.dot_general(eye, x[:, CH * c:CH * (c + 1)], dn)
            h, row = divmod(c, 512 // (2 * CH))
            out_ref[pl.ds(CH * row, CH), pl.ds(h * D, D)] = part
        out_ref[pl.ds(256, 32), :] = tp_ref[...]


def _to_pair_table(table_t, tail_pairs):
    return pl.pallas_call(
        _transpose_block,
        grid=(TB,),
        in_specs=[
            pl.BlockSpec(memory_space=pl.ANY),
            pl.BlockSpec((32, 2 * D), lambda i: (0, 0)),
        ],
        out_specs=pl.BlockSpec((TW // 2, 2 * D), lambda i: (i, 0)),
        out_shape=jax.ShapeDtypeStruct((VP, 2 * D), jnp.float32),
        scratch_shapes=[
            pltpu.VMEM((2, D, TW), jnp.float32),
            pltpu.SemaphoreType.DMA((2,)),
        ],
    )(table_t, tail_pairs)


@functools.partial(
    pl.kernel,
    out_type=[
        jax.ShapeDtypeStruct((B,), jnp.float32),
        jax.ShapeDtypeStruct((NEG * B,), jnp.float32),
    ],
    mesh=plsc.VectorSubcoreMesh(core_axis_name="c", subcore_axis_name="s"),
    compiler_params=pltpu.CompilerParams(needs_layout_passes=False),
    scratch_types=[
        pltpu.VMEM((C,), jnp.int32),
        pltpu.VMEM((C,), jnp.int32),
        pltpu.VMEM((C * NEG,), jnp.int32),
        pltpu.VMEM((C,), jnp.int32),
        pltpu.VMEM((C,), jnp.int32),
        pltpu.VMEM((C * NEG,), jnp.int32),
        pltpu.VMEM((C,), jnp.int32),
        pltpu.VMEM((C,), jnp.int32),
        pltpu.VMEM((C * NEG,), jnp.int32),
        pltpu.VMEM((C, 2 * D), jnp.float32),
        pltpu.VMEM((C, 2 * D), jnp.float32),
        pltpu.VMEM((C * NEG, 2 * D), jnp.float32),
        pltpu.VMEM((C,), jnp.float32),
        pltpu.VMEM((NEG, C), jnp.float32),
        pltpu.SemaphoreType.DMA,
    ],
)
def _sc_scores(pos_u, pos_v, neg_flat, u_pair, v_pair, out_pos, out_neg,
               idx_u, idx_v, idx_n, pair_u, pair_v, pair_n,
               off_u, off_v, off_n, u_rows, v_rows, n_rows,
               acc_p, acc_n, sem):
    wid = lax.axis_index("s") * NC + lax.axis_index("c")
    iota = lax.iota(jnp.int32, L)

    def round_body(r, _):
        base = wid * BPW + r * C
        pltpu.sync_copy(pos_u.at[pl.ds(base, C)], idx_u)
        pltpu.sync_copy(pos_v.at[pl.ds(base, C)], idx_v)
        pltpu.sync_copy(neg_flat.at[pl.ds(base * NEG, C * NEG)], idx_n)

        def pairsplit(i):
            pm = ((i >> 14) << 13) | (i & 8191)
            l = i - MAIN
            q = i - T1
            pp = jnp.where(i < MAIN, pm,
                           jnp.where(i < T1, MP + (l & 255), P2 + (q & 31)))
            ob = jnp.where(i < MAIN, (i >> 13) & 1,
                           jnp.where(i < T1, (l >> 8) & 1, (q >> 5) & 1))
            return pp, ob * D

        def shift_body(t, _):
            sl = pl.ds(t * L, L)
            pair_u[sl], off_u[sl] = pairsplit(idx_u[sl])
            pair_v[sl], off_v[sl] = pairsplit(idx_v[sl])
            return 0

        lax.fori_loop(0, C // L, shift_body, 0)

        def shift_n_body(t, _):
            sl = pl.ds(t * L, L)
            pair_n[sl], off_n[sl] = pairsplit(idx_n[sl])
            return 0

        lax.fori_loop(0, C * NEG // L, shift_n_body, 0)

        cu = pltpu.async_copy(u_pair.at[pair_u], u_rows, sem)
        cv = pltpu.async_copy(v_pair.at[pair_v], v_rows, sem)
        cn = pltpu.async_copy(v_pair.at[pair_n], n_rows, sem)
        cu.wait()
        cv.wait()
        cn.wait()

        def dot_bcast(u, w):
            p = u[0] * w[0]
            for kk in range(1, NV):
                p = p + u[kk] * w[kk]
            return jnp.full((L,), jnp.sum(p), jnp.float32)

        def load_row(ref, row, off):
            return [ref[row, pl.ds(off + L * kk, L)] for kk in range(NV)]

        def gbody(g, _):
            ou = off_u[pl.ds(g * L, L)]
            ov = off_v[pl.ds(g * L, L)]
            onn = [off_n[pl.ds(g * L * NEG + L * m, L)] for m in range(NEG)]
            accs = [jnp.zeros((L,), jnp.float32)] * (1 + NEG)
            for j in range(L):
                i = g * L + j
                lane = jnp.equal(iota, j)
                u = load_row(u_rows, i, ou[j])
                v = load_row(v_rows, i, ov[j])
                accs[0] = jnp.where(lane, dot_bcast(u, v), accs[0])
                for n in range(NEG):
                    k = j * NEG + n
                    w = load_row(n_rows, i * NEG + n, onn[k // L][k % L])
                    accs[1 + n] = jnp.where(lane, dot_bcast(u, w),
                                            accs[1 + n])
            acc_p[pl.ds(g * L, L)] = accs[0]
            for n in range(NEG):
                acc_n[n, pl.ds(g * L, L)] = accs[1 + n]
            return 0

        lax.fori_loop(0, C // L, gbody, 0)

        pltpu.sync_copy(acc_p, out_pos.at[pl.ds(base, C)])
        for n in range(NEG):
            pltpu.sync_copy(acc_n.at[n], out_neg.at[pl.ds(n * B + base, C)])
        return 0

    lax.fori_loop(0, R, round_body, 0)


def _tc_loss_kernel(p_ref, n_ref, o_ref):
    p = jnp.clip(p_ref[...], -10.0, 10.0)
    n = jnp.clip(n_ref[...], -10.0, 10.0)
    loss_pos = jnp.log1p(jnp.exp(-p))
    loss_neg = jnp.log1p(jnp.exp(n))
    o_ref[0, 0] = (jnp.sum(loss_pos) + jnp.sum(loss_neg)) * np.float32(1.0 / B)


def kernel(pos_u, pos_v, neg_v, u_table, v_table):
    pos_u = pos_u.astype(jnp.int32)
    pos_v = pos_v.astype(jnp.int32)
    neg_flat = neg_v.reshape(-1).astype(jnp.int32)

    ut = u_table[T1:]
    vt = v_table[T1:]
    u_pair = _to_pair_table(
        u_table.T, jnp.concatenate([ut[0:32], ut[32:64]], axis=1))
    v_pair = _to_pair_table(
        v_table.T, jnp.concatenate([vt[0:32], vt[32:64]], axis=1))

    dots_pos, dots_neg = _sc_scores(pos_u, pos_v, neg_flat, u_pair, v_pair)

    out = pl.pallas_call(
        _tc_loss_kernel,
        out_shape=jax.ShapeDtypeStruct((1, 1), jnp.float32),
        out_specs=pl.BlockSpec(memory_space=pltpu.SMEM),
    )(dots_pos.reshape(B // 128, 128), dots_neg.reshape(B * NEG // 128, 128))
    return out[0, 0]
